# baseline (speedup 1.0000x reference)
import functools

import jax
import jax.numpy as jnp
from jax import lax
from jax.experimental import pallas as pl
from jax.experimental.pallas import tpu as pltpu
from jax.experimental.pallas import tpu_sc as plsc

N = 50000
E = 800000
E_DIM = 64
B_DIM = 32
S_DIM = 32

BATCH = 128
NSLOT = 4
NIB = 8
NB_TOT = E // BATCH
NTILES = 16
NB_PER_TILE = -(-NB_TOT // NTILES)
ROWS_PER_TILE = 3128
RCHUNK = 184
NCHUNK = ROWS_PER_TILE // RCHUNK


def _make_sc_agg(with_deg: bool):
    mesh = plsc.VectorSubcoreMesh(core_axis_name="c", subcore_axis_name="s")

    out_type = [jax.ShapeDtypeStruct((N, 32), jnp.float32) for _ in range(4)]
    if with_deg:
        out_type.append(jax.ShapeDtypeStruct((N,), jnp.float32))

    scratch = dict(
        acc=pltpu.VMEM_SHARED((N, 32), jnp.float32),
        zb2=pltpu.VMEM((RCHUNK, 32), jnp.float32),
    )
    for j in range(NSLOT):
        scratch[f"rw{j}"] = pltpu.VMEM((BATCH, 32), jnp.float32)
        scratch[f"gsem{j}"] = pltpu.SemaphoreType.DMA
        scratch[f"ssem{j}"] = pltpu.SemaphoreType.DMA
    for r in range(NIB):
        scratch[f"ib{r}"] = pltpu.VMEM((2, BATCH), jnp.int32)
        scratch[f"sx{r}"] = pltpu.VMEM((BATCH,), jnp.int32)
        scratch[f"isem{r}"] = pltpu.SemaphoreType.DMA
    if with_deg:
        scratch["dega"] = pltpu.VMEM_SHARED((N,), jnp.float32)
        scratch["ones"] = pltpu.VMEM((BATCH,), jnp.float32)
        scratch["zb1"] = pltpu.VMEM((RCHUNK,), jnp.float32)

    def body(feat, ei, z2d, *rest, **sc):
        if with_deg:
            z1d = rest[0]
            o0, o1, o2, o3, odeg = rest[1:6]
        else:
            o0, o1, o2, o3 = rest[0:4]
        outs = (o0, o1, o2, o3)
        acc = sc["acc"]
        rw = tuple(sc[f"rw{j}"] for j in range(NSLOT))
        gsem = tuple(sc[f"gsem{j}"] for j in range(NSLOT))
        ssem = tuple(sc[f"ssem{j}"] for j in range(NSLOT))
        ib = tuple(sc[f"ib{r}"] for r in range(NIB))
        sx = tuple(sc[f"sx{r}"] for r in range(NIB))
        isem = tuple(sc[f"isem{r}"] for r in range(NIB))

        core = lax.axis_index("c")
        t = lax.axis_index("s")
        roff = jnp.minimum(t * ROWS_PER_TILE, N - ROWS_PER_TILE)

        if with_deg:
            ones = sc["ones"]
            for g in range(BATCH // 16):
                ones[pl.ds(g * 16, 16)] = jnp.ones((16,), jnp.float32)

        def fire_idx(r, k):
            base = (k * NTILES + t) * BATCH
            pltpu.async_copy(ei.at[:, pl.ds(base, BATCH)], ib[r], isem[r])

        def wait_idx_scale(r, k, cc):
            base = (k * NTILES + t) * BATCH
            pltpu.make_async_copy(ei.at[:, pl.ds(base, BATCH)], ib[r],
                                  isem[r]).wait()
            for g in range(BATCH // 16):
                v = ib[r][0, pl.ds(g * 16, 16)]
                sx[r][pl.ds(g * 16, 16)] = v * 4 + cc

        def valid(k):
            return (k * NTILES + t) < NB_TOT

        def run_pass(p, out, do_deg):
            cc = core * 2 + p
            pltpu.sync_copy(z2d.at[pl.ds(0, RCHUNK)], sc["zb2"])
            if do_deg:
                pltpu.sync_copy(z1d.at[pl.ds(0, RCHUNK)], sc["zb1"])
            for i in range(NCHUNK):
                pltpu.sync_copy(sc["zb2"],
                                acc.at[pl.ds(roff + i * RCHUNK, RCHUNK)])
                if do_deg:
                    pltpu.sync_copy(
                        sc["zb1"],
                        sc["dega"].at[pl.ds(roff + i * RCHUNK, RCHUNK)])
            plsc.subcore_barrier()

            fire_idx(0, 0)
            pl.when(valid(1))(lambda: fire_idx(1, 1))
            pl.when(valid(2))(lambda: fire_idx(2, 2))
            wait_idx_scale(0, 0, cc)
            pltpu.async_copy(feat.at[sx[0]], rw[0], gsem[0])

            def prime1():
                wait_idx_scale(1, 1, cc)
                pltpu.async_copy(feat.at[sx[1]], rw[1], gsem[1])

            pl.when(valid(1))(prime1)

            def step(k, j, r):
                j2 = (j + 2) % NSLOT
                r2 = (r + 2) % NIB
                r3 = (r + 3) % NIB

                def wait_scat():
                    pltpu.make_async_copy(
                        rw[j2], acc.at[ib[(r - 2) % NIB].at[1]],
                        ssem[j2]).wait()

                pl.when((k >= 2) & valid(k - 2))(wait_scat)

                def fire_gather():
                    wait_idx_scale(r2, k + 2, cc)
                    pltpu.async_copy(feat.at[sx[r2]], rw[j2], gsem[j2])

                pl.when(valid(k + 2))(fire_gather)

                pl.when(valid(k + 3))(lambda: fire_idx(r3, k + 3))

                def do_scatter():
                    pltpu.make_async_copy(feat.at[sx[r]], rw[j],
                                          gsem[j]).wait()
                    pltpu.async_copy(rw[j], acc.at[ib[r].at[1]], ssem[j],
                                     add=True)
                    if do_deg:
                        pltpu.sync_copy(sc["ones"],
                                        sc["dega"].at[ib[r].at[1]],
                                        add=True)

                pl.when(valid(k))(do_scatter)

            def loop_body(k, carry):
                for r in range(NIB):
                    pl.when(k % NIB == r)(
                        functools.partial(step, k, r % NSLOT, r))
                return carry

            lax.fori_loop(0, NB_PER_TILE, loop_body, 0)

            for d in (2, 1):
                kk = NB_PER_TILE - d
                pl.when(valid(kk))(
                    lambda kk=kk: pltpu.make_async_copy(
                        rw[kk % NSLOT], acc.at[ib[kk % NIB].at[1]],
                        ssem[kk % NSLOT]).wait())
            plsc.subcore_barrier()

            for i in range(NCHUNK):
                off = roff + i * RCHUNK
                pltpu.sync_copy(acc.at[pl.ds(off, RCHUNK)], sc["zb2"])
                pltpu.sync_copy(sc["zb2"], out.at[pl.ds(off, RCHUNK)])
                if do_deg:
                    pltpu.sync_copy(sc["dega"].at[pl.ds(off, RCHUNK)],
                                    sc["zb1"])
                    pltpu.sync_copy(sc["zb1"], odeg.at[pl.ds(off, RCHUNK)])
            plsc.subcore_barrier()

        def core0():
            run_pass(0, o0, with_deg)
            run_pass(1, o1, False)

        def core1():
            run_pass(0, o2, False)
            run_pass(1, o3, False)

        pl.when(core == 0)(core0)
        pl.when(core == 1)(core1)

    return pl.kernel(body, out_type=out_type, mesh=mesh,
                     scratch_types=scratch,
                     compiler_params=pltpu.CompilerParams(
                         use_tc_tiling_on_sc=False))


_sc_agg_deg = _make_sc_agg(True)
_sc_agg = _make_sc_agg(False)


R = 1000
GRID = N // R

_f32 = jnp.float32


def _dot(x, w):
    return lax.dot_general(x, w, (((1,), (1,)), ((), ())),
                           preferred_element_type=_f32)


def _leaky(x):
    return jnp.where(x >= 0, x, 0.2 * x)


def _rsum(x):
    ones = jnp.ones((x.shape[1], 1), _f32)
    return lax.dot_general(x * x, ones, (((1,), (0,)), ((), ())),
                           precision=lax.Precision.HIGHEST,
                           preferred_element_type=_f32)


def _log0_scale(b, scb):
    bn = jnp.sqrt(_rsum(b))
    x = scb * bn
    at = 0.5 * jnp.log((1.0 + x) / (1.0 - x))
    return (2.0 / scb) * at / bn * b


def _exp0(v, scb):
    vn = jnp.sqrt(_rsum(v))
    return jnp.tanh(scb * vn / 2.0) * v / (scb * vn)


def _l2n(x):
    n = jnp.sqrt(_rsum(x))
    return x / jnp.maximum(n, 1e-12)


def _pre_kernel(e_ref, b_ref, s_ref, we, wb, ws, be, bb, bs, scb_ref, h):
    scb = scb_ref[0, 0]
    te = _dot(e_ref[...], we[...]) + be[...]
    tang = _log0_scale(b_ref[...], scb)
    tb = _dot(tang, wb[...]) + bb[...]
    ns = _l2n(s_ref[...])
    ts = _l2n(_dot(ns, ws[...]) + bs[...])
    h[...] = jnp.concatenate([te, tb, ts], axis=1)


def _mid_kernel(a0, a1, a2, a3, deg, we, wb, ws, be, bb, bs, scb_ref, h):
    scb = scb_ref[0, 0]
    inv = 1.0 / jnp.maximum(deg[...], 1.0)
    e1 = _leaky(jnp.concatenate([a0[...], a1[...]], axis=1) * inv)
    b1 = _exp0(a2[...] * inv, scb)
    s1 = _l2n(a3[...] * inv)
    te = _dot(e1, we[...]) + be[...]
    tang = _log0_scale(b1, scb)
    tb = _dot(tang, wb[...]) + bb[...]
    ns = _l2n(s1)
    ts = _l2n(_dot(ns, ws[...]) + bs[...])
    h[...] = jnp.concatenate([te, tb, ts], axis=1)


def _post_kernel(a0, a1, a2, a3, deg, scb_ref, eo, bo, so):
    scb = scb_ref[0, 0]
    inv = 1.0 / jnp.maximum(deg[...], 1.0)
    eo[...] = _leaky(jnp.concatenate([a0[...], a1[...]], axis=1) * inv)
    bo[...] = _exp0(a2[...] * inv, scb)
    so[...] = _l2n(a3[...] * inv)


def _rows(shape):
    return pl.BlockSpec((R,) + shape[1:], lambda i: (i,) + (0,) * (len(shape) - 1))


def _full(shape):
    return pl.BlockSpec(shape, lambda i: (0,) * len(shape))


def _tc_pre(e, b, s, we, wb, ws, be, bb, bs, scb):
    return pl.pallas_call(
        _pre_kernel,
        grid=(GRID,),
        in_specs=[_rows((N, E_DIM)), _rows((N, B_DIM)), _rows((N, S_DIM)),
                  _full((E_DIM, E_DIM)), _full((B_DIM, B_DIM)),
                  _full((S_DIM, S_DIM)),
                  _full((1, E_DIM)), _full((1, B_DIM)), _full((1, S_DIM)),
                  _full((1, 1))],
        out_specs=_rows((N, 128)),
        out_shape=jax.ShapeDtypeStruct((N, 128), _f32),
    )(e, b, s, we, wb, ws, be, bb, bs, scb)


def _tc_mid(a0, a1, a2, a3, deg, we, wb, ws, be, bb, bs, scb):
    return pl.pallas_call(
        _mid_kernel,
        grid=(GRID,),
        in_specs=[_rows((N, 32))] * 4 + [_rows((N, 1)),
                  _full((E_DIM, E_DIM)), _full((B_DIM, B_DIM)),
                  _full((S_DIM, S_DIM)),
                  _full((1, E_DIM)), _full((1, B_DIM)), _full((1, S_DIM)),
                  _full((1, 1))],
        out_specs=_rows((N, 128)),
        out_shape=jax.ShapeDtypeStruct((N, 128), _f32),
    )(a0, a1, a2, a3, deg, we, wb, ws, be, bb, bs, scb)


def _tc_post(a0, a1, a2, a3, deg, scb):
    return pl.pallas_call(
        _post_kernel,
        grid=(GRID,),
        in_specs=[_rows((N, 32))] * 4 + [_rows((N, 1)), _full((1, 1))],
        out_specs=[_rows((N, E_DIM)), _rows((N, B_DIM)), _rows((N, S_DIM))],
        out_shape=[jax.ShapeDtypeStruct((N, E_DIM), _f32),
                   jax.ShapeDtypeStruct((N, B_DIM), _f32),
                   jax.ShapeDtypeStruct((N, S_DIM), _f32)],
    )(a0, a1, a2, a3, deg, scb)


def kernel(e_emb, b_emb, s_emb, b_curvature, s_curvature,
           We0, be0, Wb0, bb0, Ws0, bs0,
           We1, be1, Wb1, bb1, Ws1, bs1, edge_index):
    z2d = jnp.zeros((N, 32), _f32)
    z1d = jnp.zeros((N,), _f32)
    scb = jnp.sqrt(b_curvature).reshape(1, 1)

    h = _tc_pre(e_emb, b_emb, s_emb, We0, Wb0, Ws0,
                be0.reshape(1, -1), bb0.reshape(1, -1), bs0.reshape(1, -1),
                scb)
    a0, a1, a2, a3, deg = _sc_agg_deg(h.reshape(4 * N, 32), edge_index,
                                      z2d, z1d)
    deg2 = deg.reshape(N, 1)
    h = _tc_mid(a0, a1, a2, a3, deg2, We1, Wb1, Ws1,
                be1.reshape(1, -1), bb1.reshape(1, -1), bs1.reshape(1, -1),
                scb)
    t0, t1, t2, t3 = _sc_agg(h.reshape(4 * N, 32), edge_index, z2d)
    return _tc_post(t0, t1, t2, t3, deg2, scb)


# speedup vs baseline: 1.2823x; 1.2823x over previous
import functools

import jax
import jax.numpy as jnp
from jax import lax
from jax.experimental import pallas as pl
from jax.experimental.pallas import tpu as pltpu
from jax.experimental.pallas import tpu_sc as plsc

N = 50000
E = 800000
E_DIM = 64
B_DIM = 32
S_DIM = 32

BATCH = 128
NSLOT = 4
NIB = 8
NB_TOT = E // BATCH
NTILES = 16
NB_PER_TILE = -(-NB_TOT // NTILES)
ROWS_PER_TILE = 3128
RCHUNK = 184
NCHUNK = ROWS_PER_TILE // RCHUNK


def _make_sc_agg(with_deg: bool):
    mesh = plsc.VectorSubcoreMesh(core_axis_name="c", subcore_axis_name="s")

    out_type = [jax.ShapeDtypeStruct((N, 32), jnp.float32) for _ in range(4)]
    if with_deg:
        out_type.append(jax.ShapeDtypeStruct((N,), jnp.float32))

    scratch = dict(
        acc=pltpu.VMEM_SHARED((N, 32), jnp.float32),
        zb2=pltpu.VMEM((RCHUNK, 32), jnp.float32),
    )
    for j in range(NSLOT):
        scratch[f"rw{j}"] = pltpu.VMEM((BATCH, 32), jnp.float32)
        scratch[f"gsem{j}"] = pltpu.SemaphoreType.DMA
        scratch[f"ssem{j}"] = pltpu.SemaphoreType.DMA
    for r in range(NIB):
        scratch[f"ib{r}"] = pltpu.VMEM((2, BATCH), jnp.int32)
        scratch[f"sx{r}"] = pltpu.VMEM((BATCH,), jnp.int32)
        scratch[f"isem{r}"] = pltpu.SemaphoreType.DMA
    if with_deg:
        scratch["dega"] = pltpu.VMEM_SHARED((N,), jnp.float32)
        scratch["ones"] = pltpu.VMEM((BATCH,), jnp.float32)
        scratch["zb1"] = pltpu.VMEM((RCHUNK,), jnp.float32)

    def body(feat, ei, z2d, *rest, **sc):
        if with_deg:
            z1d = rest[0]
            o0, o1, o2, o3, odeg = rest[1:6]
        else:
            o0, o1, o2, o3 = rest[0:4]
        outs = (o0, o1, o2, o3)
        acc = sc["acc"]
        rw = tuple(sc[f"rw{j}"] for j in range(NSLOT))
        gsem = tuple(sc[f"gsem{j}"] for j in range(NSLOT))
        ssem = tuple(sc[f"ssem{j}"] for j in range(NSLOT))
        ib = tuple(sc[f"ib{r}"] for r in range(NIB))
        sx = tuple(sc[f"sx{r}"] for r in range(NIB))
        isem = tuple(sc[f"isem{r}"] for r in range(NIB))

        core = lax.axis_index("c")
        t = lax.axis_index("s")
        roff = jnp.minimum(t * ROWS_PER_TILE, N - ROWS_PER_TILE)

        if with_deg:
            ones = sc["ones"]
            for g in range(BATCH // 16):
                ones[pl.ds(g * 16, 16)] = jnp.ones((16,), jnp.float32)

        def fire_idx(r, k):
            base = (k * NTILES + t) * BATCH
            pltpu.async_copy(ei.at[:, pl.ds(base, BATCH)], ib[r], isem[r])

        def wait_idx_scale(r, k, cc):
            base = (k * NTILES + t) * BATCH
            pltpu.make_async_copy(ei.at[:, pl.ds(base, BATCH)], ib[r],
                                  isem[r]).wait()
            for g in range(BATCH // 16):
                v = ib[r][0, pl.ds(g * 16, 16)]
                sx[r][pl.ds(g * 16, 16)] = v * 4 + cc

        def valid(k):
            return (k * NTILES + t) < NB_TOT

        def run_pass(p, out, do_deg):
            cc = core * 2 + p
            pltpu.sync_copy(z2d.at[pl.ds(0, RCHUNK)], sc["zb2"])
            if do_deg:
                pltpu.sync_copy(z1d.at[pl.ds(0, RCHUNK)], sc["zb1"])
            for i in range(NCHUNK):
                pltpu.sync_copy(sc["zb2"],
                                acc.at[pl.ds(roff + i * RCHUNK, RCHUNK)])
                if do_deg:
                    pltpu.sync_copy(
                        sc["zb1"],
                        sc["dega"].at[pl.ds(roff + i * RCHUNK, RCHUNK)])
            plsc.subcore_barrier()

            fire_idx(0, 0)
            pl.when(valid(1))(lambda: fire_idx(1, 1))
            pl.when(valid(2))(lambda: fire_idx(2, 2))
            wait_idx_scale(0, 0, cc)
            pltpu.async_copy(feat.at[sx[0]], rw[0], gsem[0])

            def prime1():
                wait_idx_scale(1, 1, cc)
                pltpu.async_copy(feat.at[sx[1]], rw[1], gsem[1])

            pl.when(valid(1))(prime1)

            def step(k, j, r):
                j2 = (j + 2) % NSLOT
                r2 = (r + 2) % NIB
                r3 = (r + 3) % NIB

                def wait_scat():
                    pltpu.make_async_copy(
                        rw[j2], acc.at[ib[(r - 2) % NIB].at[1]],
                        ssem[j2]).wait()

                pl.when((k >= 2) & valid(k - 2))(wait_scat)

                def fire_gather():
                    wait_idx_scale(r2, k + 2, cc)
                    pltpu.async_copy(feat.at[sx[r2]], rw[j2], gsem[j2])

                pl.when(valid(k + 2))(fire_gather)

                pl.when(valid(k + 3))(lambda: fire_idx(r3, k + 3))

                def do_scatter():
                    pltpu.make_async_copy(feat.at[sx[r]], rw[j],
                                          gsem[j]).wait()
                    pltpu.async_copy(rw[j], acc.at[ib[r].at[1]], ssem[j],
                                     add=True)
                    if do_deg:
                        pltpu.sync_copy(sc["ones"],
                                        sc["dega"].at[ib[r].at[1]],
                                        add=True)

                pl.when(valid(k))(do_scatter)

            def loop_body(k, carry):
                for r in range(NIB):
                    pl.when(k % NIB == r)(
                        functools.partial(step, k, r % NSLOT, r))
                return carry

            lax.fori_loop(0, NB_PER_TILE, loop_body, 0)

            for d in (2, 1):
                kk = NB_PER_TILE - d
                pl.when(valid(kk))(
                    lambda kk=kk: pltpu.make_async_copy(
                        rw[kk % NSLOT], acc.at[ib[kk % NIB].at[1]],
                        ssem[kk % NSLOT]).wait())
            plsc.subcore_barrier()

            for i in range(NCHUNK):
                off = roff + i * RCHUNK
                pltpu.sync_copy(acc.at[pl.ds(off, RCHUNK)], sc["zb2"])
                pltpu.sync_copy(sc["zb2"], out.at[pl.ds(off, RCHUNK)])
                if do_deg:
                    pltpu.sync_copy(sc["dega"].at[pl.ds(off, RCHUNK)],
                                    sc["zb1"])
                    pltpu.sync_copy(sc["zb1"], odeg.at[pl.ds(off, RCHUNK)])
            plsc.subcore_barrier()

        def core0():
            run_pass(0, o0, with_deg)
            run_pass(1, o1, False)

        def core1():
            run_pass(0, o2, False)
            run_pass(1, o3, False)

        pl.when(core == 0)(core0)
        pl.when(core == 1)(core1)

    return pl.kernel(body, out_type=out_type, mesh=mesh,
                     scratch_types=scratch,
                     compiler_params=pltpu.CompilerParams(
                         use_tc_tiling_on_sc=False))


_sc_agg_deg = _make_sc_agg(True)
_sc_agg = _make_sc_agg(False)


R = 1000
GRID = N // R

_f32 = jnp.float32


def _dot(x, w):
    return lax.dot_general(x, w, (((1,), (1,)), ((), ())),
                           preferred_element_type=_f32)


def _leaky(x):
    return jnp.where(x >= 0, x, 0.2 * x)


def _log0_scale(b, scb):
    bn = jnp.sqrt(jnp.sum(b * b, axis=1, keepdims=True))
    x = scb * bn
    at = 0.5 * jnp.log((1.0 + x) / (1.0 - x))
    return (2.0 / scb) * at / bn * b


def _exp0(v, scb):
    vn = jnp.sqrt(jnp.sum(v * v, axis=1, keepdims=True))
    return jnp.tanh(scb * vn / 2.0) * v / (scb * vn)


def _l2n(x):
    n = jnp.sqrt(jnp.sum(x * x, axis=1, keepdims=True))
    return x / jnp.maximum(n, 1e-12)


def _pre_kernel(e_ref, b_ref, s_ref, we, wb, ws, be, bb, bs, scb_ref, h):
    scb = scb_ref[0, 0]
    te = _dot(e_ref[...], we[...]) + be[...]
    tang = _log0_scale(b_ref[...], scb)
    tb = _dot(tang, wb[...]) + bb[...]
    ns = _l2n(s_ref[...])
    ts = _l2n(_dot(ns, ws[...]) + bs[...])
    h[...] = jnp.concatenate([te, tb, ts], axis=1)


def _mid_kernel(a0, a1, a2, a3, deg, we, wb, ws, be, bb, bs, scb_ref, h):
    scb = scb_ref[0, 0]
    inv = 1.0 / jnp.maximum(deg[...], 1.0)
    e1 = _leaky(jnp.concatenate([a0[...], a1[...]], axis=1) * inv)
    b1 = _exp0(a2[...] * inv, scb)
    s1 = _l2n(a3[...] * inv)
    te = _dot(e1, we[...]) + be[...]
    tang = _log0_scale(b1, scb)
    tb = _dot(tang, wb[...]) + bb[...]
    ns = _l2n(s1)
    ts = _l2n(_dot(ns, ws[...]) + bs[...])
    h[...] = jnp.concatenate([te, tb, ts], axis=1)


def _post_kernel(a0, a1, a2, a3, deg, scb_ref, eo, bo, so):
    scb = scb_ref[0, 0]
    inv = 1.0 / jnp.maximum(deg[...], 1.0)
    eo[...] = _leaky(jnp.concatenate([a0[...], a1[...]], axis=1) * inv)
    bo[...] = _exp0(a2[...] * inv, scb)
    so[...] = _l2n(a3[...] * inv)


def _rows(shape):
    return pl.BlockSpec((R,) + shape[1:], lambda i: (i,) + (0,) * (len(shape) - 1))


def _full(shape):
    return pl.BlockSpec(shape, lambda i: (0,) * len(shape))


def _tc_pre(e, b, s, we, wb, ws, be, bb, bs, scb):
    return pl.pallas_call(
        _pre_kernel,
        grid=(GRID,),
        in_specs=[_rows((N, E_DIM)), _rows((N, B_DIM)), _rows((N, S_DIM)),
                  _full((E_DIM, E_DIM)), _full((B_DIM, B_DIM)),
                  _full((S_DIM, S_DIM)),
                  _full((1, E_DIM)), _full((1, B_DIM)), _full((1, S_DIM)),
                  _full((1, 1))],
        out_specs=_rows((N, 128)),
        out_shape=jax.ShapeDtypeStruct((N, 128), _f32),
    )(e, b, s, we, wb, ws, be, bb, bs, scb)


def _tc_mid(a0, a1, a2, a3, deg, we, wb, ws, be, bb, bs, scb):
    return pl.pallas_call(
        _mid_kernel,
        grid=(GRID,),
        in_specs=[_rows((N, 32))] * 4 + [_rows((N, 1)),
                  _full((E_DIM, E_DIM)), _full((B_DIM, B_DIM)),
                  _full((S_DIM, S_DIM)),
                  _full((1, E_DIM)), _full((1, B_DIM)), _full((1, S_DIM)),
                  _full((1, 1))],
        out_specs=_rows((N, 128)),
        out_shape=jax.ShapeDtypeStruct((N, 128), _f32),
    )(a0, a1, a2, a3, deg, we, wb, ws, be, bb, bs, scb)


def _tc_post(a0, a1, a2, a3, deg, scb):
    return pl.pallas_call(
        _post_kernel,
        grid=(GRID,),
        in_specs=[_rows((N, 32))] * 4 + [_rows((N, 1)), _full((1, 1))],
        out_specs=[_rows((N, E_DIM)), _rows((N, B_DIM)), _rows((N, S_DIM))],
        out_shape=[jax.ShapeDtypeStruct((N, E_DIM), _f32),
                   jax.ShapeDtypeStruct((N, B_DIM), _f32),
                   jax.ShapeDtypeStruct((N, S_DIM), _f32)],
    )(a0, a1, a2, a3, deg, scb)


def kernel(e_emb, b_emb, s_emb, b_curvature, s_curvature,
           We0, be0, Wb0, bb0, Ws0, bs0,
           We1, be1, Wb1, bb1, Ws1, bs1, edge_index):
    z2d = jnp.zeros((N, 32), _f32)
    z1d = jnp.zeros((N,), _f32)
    scb = jnp.sqrt(b_curvature).reshape(1, 1)

    h = _tc_pre(e_emb, b_emb, s_emb, We0, Wb0, Ws0,
                be0.reshape(1, -1), bb0.reshape(1, -1), bs0.reshape(1, -1),
                scb)
    a0, a1, a2, a3, deg = _sc_agg_deg(h.reshape(4 * N, 32), edge_index,
                                      z2d, z1d)
    deg2 = deg.reshape(N, 1)
    h = _tc_mid(a0, a1, a2, a3, deg2, We1, Wb1, Ws1,
                be1.reshape(1, -1), bb1.reshape(1, -1), bs1.reshape(1, -1),
                scb)
    t0, t1, t2, t3 = _sc_agg(h.reshape(4 * N, 32), edge_index, z2d)
    return _tc_post(t0, t1, t2, t3, deg2, scb)
